# Initial kernel scaffold; baseline (speedup 1.0000x reference)
import jax
import jax.numpy as jnp
from jax.experimental import pallas as pl


def kernel(feat, shuf_feat, aug1_feat, aug2_feat, adj_index, adj_weight, aug1_index, aug1_weight, aug2_index, aug2_weight, W1, b1, W2, b2):
    raise NotImplementedError("write your pallas kernel here")


# speedup vs baseline: 2.2933x; 2.2933x over previous
import functools

import jax
import jax.numpy as jnp
from jax import lax
from jax.experimental import pallas as pl
from jax.experimental.pallas import tpu as pltpu
from jax.experimental.pallas import tpu_sc as plsc

_NC = 2
_NS = 16
_NW = _NC * _NS
_CHUNK = 128


def _prep_edges(idx, w):
    e = w.shape[0]
    epw = -(-e // _NW)
    nch = -(-epw // _CHUNK)
    nch = -(-nch // 8) * 8
    pad = _NW * nch * _CHUNK - e
    src = jnp.pad(idx[1], (0, pad)).reshape(_NW, nch, _CHUNK)
    dst = jnp.pad(idx[0], (0, pad)).reshape(_NW, nch, _CHUNK)
    ww = jnp.pad(w, (0, pad)).reshape(_NW, nch, _CHUNK)
    return src, dst, ww


@functools.lru_cache(maxsize=None)
def _spmm_kernel(n, c, nch):
    npad = -(-n // 128) * 128
    rpt = npad // _NS
    ck = _CHUNK
    nvec = c // 16
    mesh = plsc.VectorSubcoreMesh(core_axis_name="c", subcore_axis_name="s")

    @functools.partial(
        pl.kernel,
        mesh=mesh,
        out_type=jax.ShapeDtypeStruct((_NC, npad, c), jnp.float32),
        scratch_types=[
            pltpu.VMEM((nch, ck), jnp.int32),
            pltpu.VMEM((nch, ck), jnp.int32),
            pltpu.VMEM((nch, ck), jnp.float32),
            pltpu.VMEM((ck, c), jnp.float32),
            pltpu.VMEM_SHARED((npad, c), jnp.float32),
            pltpu.SemaphoreType.DMA,
        ],
    )
    def k(x_hbm, src_hbm, dst_hbm, w_hbm, out_hbm, srcv, dstv, wv, rows, accum, sem):
        ci = lax.axis_index("c")
        si = lax.axis_index("s")
        wid = si * _NC + ci
        pltpu.sync_copy(src_hbm.at[wid], srcv)
        pltpu.sync_copy(dst_hbm.at[wid], dstv)
        pltpu.sync_copy(w_hbm.at[wid], wv)

        zero = jnp.zeros((16,), jnp.float32)

        def zrow(i, carry):
            for cc in range(nvec):
                rows[i, pl.ds(cc * 16, 16)] = zero
            return carry

        lax.fori_loop(0, ck, zrow, 0)
        base = si * rpt
        off = 0
        while off < rpt:
            sz = min(ck, rpt - off)
            pltpu.sync_copy(rows.at[pl.ds(0, sz)], accum.at[pl.ds(base + off, sz)])
            off += sz
        plsc.subcore_barrier()

        def body(j, carry):
            pltpu.async_copy(x_hbm.at[srcv.at[j]], rows, sem).wait()

            def egroup(g, carry2):
                w16 = wv[j, pl.ds(g * 16, 16)]
                for l in range(16):
                    wl = w16[l]
                    e2 = g * 16 + l
                    for cc in range(nvec):
                        sl = pl.ds(cc * 16, 16)
                        rows[e2, sl] = rows[e2, sl] * wl
                return carry2

            lax.fori_loop(0, ck // 16, egroup, 0)
            pltpu.sync_copy(rows, accum.at[dstv.at[j]], add=True)
            return carry

        lax.fori_loop(0, nch, body, 0)
        plsc.subcore_barrier()

        off = 0
        while off < rpt:
            sz = min(ck, rpt - off)
            pltpu.sync_copy(accum.at[pl.ds(base + off, sz)],
                            out_hbm.at[ci, pl.ds(base + off, sz)])
            off += sz

    return k


def _spmm_partial(x, src, dst, w):
    n, c = x.shape
    return _spmm_kernel(n, c, src.shape[1])(x, src, dst, w)


def _dense4(gp, w1, b1, w2, n):
    c = gp.shape[-1]
    h1 = w1.shape[1]
    bn = 1000

    def body(g_ref, w1_ref, b1_ref, w2_ref, o_ref):
        g = g_ref[0, 0] + g_ref[0, 1]
        h = jnp.maximum(
            jnp.dot(g, w1_ref[...], preferred_element_type=jnp.float32)
            + b1_ref[0], 0.0)
        o_ref[0] = jnp.dot(h, w2_ref[...], preferred_element_type=jnp.float32)

    return pl.pallas_call(
        body,
        grid=(4, n // bn),
        in_specs=[
            pl.BlockSpec((1, 2, bn, c), lambda k, j: (k, 0, j, 0)),
            pl.BlockSpec((c, h1), lambda k, j: (0, 0)),
            pl.BlockSpec((1, h1), lambda k, j: (0, 0)),
            pl.BlockSpec((h1, c), lambda k, j: (0, 0)),
        ],
        out_specs=pl.BlockSpec((1, bn, c), lambda k, j: (k, j, 0)),
        out_shape=jax.ShapeDtypeStruct((4, n, c), jnp.float32),
    )(gp, w1, b1.reshape(1, -1), w2)


def _bias_relu4(sp, b2, n):
    c = sp.shape[-1]
    bn = 1000

    def body(s_ref, b_ref, o_ref):
        o_ref[0] = jnp.maximum(s_ref[0, 0] + s_ref[0, 1] + b_ref[0], 0.0)

    return pl.pallas_call(
        body,
        grid=(4, n // bn),
        in_specs=[
            pl.BlockSpec((1, 2, bn, c), lambda k, j: (k, 0, j, 0)),
            pl.BlockSpec((1, c), lambda k, j: (0, 0)),
        ],
        out_specs=pl.BlockSpec((1, bn, c), lambda k, j: (k, j, 0)),
        out_shape=jax.ShapeDtypeStruct((4, n, c), jnp.float32),
    )(sp, b2.reshape(1, -1))


def kernel(feat, shuf_feat, aug1_feat, aug2_feat, adj_index, adj_weight,
           aug1_index, aug1_weight, aug2_index, aug2_weight, W1, b1, W2, b2):
    adj_e = _prep_edges(adj_index, adj_weight)
    aug1_e = _prep_edges(aug1_index, aug1_weight)

    g0 = _spmm_partial(feat, *adj_e)
    g1 = _spmm_partial(shuf_feat, *adj_e)
    g2 = _spmm_partial(feat, *aug1_e)
    g3 = _spmm_partial(aug2_feat, *adj_e)
    gp = jnp.stack([g0, g1, g2, g3])

    n = feat.shape[0]
    p = _dense4(gp, W1, b1, W2, n)

    s0 = _spmm_partial(p[0], *adj_e)
    s1 = _spmm_partial(p[1], *adj_e)
    s2 = _spmm_partial(p[2], *aug1_e)
    s3 = _spmm_partial(p[3], *adj_e)
    sp = jnp.stack([s0, s1, s2, s3])

    out = _bias_relu4(sp, b2, n)
    return (out[0], out[1], out[2], out[3])


# speedup vs baseline: 2.6622x; 1.1609x over previous
import functools

import jax
import jax.numpy as jnp
from jax import lax
from jax.experimental import pallas as pl
from jax.experimental.pallas import tpu as pltpu
from jax.experimental.pallas import tpu_sc as plsc

_NC = 2
_NS = 16
_NW = _NC * _NS
_CHUNK = 128
_NB = 2
_NI = 8
_L = _NB // 2


def _prep_edges4(idx_a, w_a, idx_b, w_b, n):
    e = w_a.shape[0]
    epw = -(-e // _NW)
    nch = -(-epw // _CHUNK)
    nch = -(-nch // _NI) * _NI
    pad = _NW * nch * _CHUNK - e

    def slab(x):
        return jnp.pad(x, (0, pad)).reshape(_NW, nch, _CHUNK)

    sa, da, wa = slab(idx_a[1]), slab(idx_a[0]), slab(w_a)
    sb, db, wb = slab(idx_b[1]), slab(idx_b[0]), slab(w_b)

    def pack(s, d):
        return jnp.stack([s, d], axis=2)

    idx4 = jnp.stack([pack(sa, da), pack(sa + n, da),
                      pack(sb + 2 * n, db), pack(sa + 3 * n, da)])
    w4 = jnp.stack([wa, wa, wb, wa])
    return idx4, w4


@functools.lru_cache(maxsize=None)
def _spmm4_kernel(n, c, nch):
    npad = -(-n // 128) * 128
    rpt = npad // _NS
    ck = _CHUNK
    nvec = c // 16
    mesh = plsc.VectorSubcoreMesh(core_axis_name="c", subcore_axis_name="s")

    @functools.partial(
        pl.kernel,
        mesh=mesh,
        out_type=jax.ShapeDtypeStruct((_NC, 4, npad, c), jnp.float32),
        scratch_types=[
            pltpu.VMEM((_NI, 2, ck), jnp.int32),
            pltpu.VMEM((_NI, ck), jnp.float32),
            pltpu.VMEM((_NB, ck, c), jnp.float32),
            pltpu.VMEM_SHARED((npad, c), jnp.float32),
        ]
        + [pltpu.SemaphoreType.DMA] * (_NB + _NB + _NI + _NI),
    )
    def k(xs_hbm, idx_hbm, w_hbm, out_hbm, idxr, wr, rows, accum, *sems):
        semg = sems[:_NB]
        semsc = sems[_NB:2 * _NB]
        semi = sems[2 * _NB:2 * _NB + _NI]
        semw = sems[2 * _NB + _NI:]
        ci = lax.axis_index("c")
        si = lax.axis_index("s")
        wid = si * _NC + ci
        base = si * rpt
        zero = jnp.zeros((16,), jnp.float32)

        def g_copy(b, ib):
            return pltpu.make_async_copy(
                xs_hbm.at[idxr.at[ib, 0]], rows.at[b], semg[b])

        def s_copy(b, ib):
            return pltpu.make_async_copy(
                rows.at[b], accum.at[idxr.at[ib, 1]], semsc[b])

        def i_copy(ib, p, j):
            return pltpu.make_async_copy(
                idx_hbm.at[p, wid, j], idxr.at[ib], semi[ib])

        def w_copy(ib, p, j):
            return pltpu.make_async_copy(
                w_hbm.at[p, wid, j], wr.at[ib], semw[ib])

        def scale(b, ib):
            def egroup(g, carry2):
                w16 = wr[ib, pl.ds(g * 16, 16)]
                for l in range(16):
                    wl = w16[l]
                    e2 = g * 16 + l
                    for cc in range(nvec):
                        sl = pl.ds(cc * 16, 16)
                        rows[b, e2, sl] = rows[b, e2, sl] * wl
                return carry2

            lax.fori_loop(0, ck // 16, egroup, 0)

        def problem(p, carry):
            for ib in range(_NI - 2):
                i_copy(ib, p, ib).start()
                w_copy(ib, p, ib).start()

            def zrow(i, cz):
                for cc in range(nvec):
                    rows[0, i, pl.ds(cc * 16, 16)] = zero
                return cz

            lax.fori_loop(0, ck, zrow, 0)
            off = 0
            while off < rpt:
                sz = min(ck, rpt - off)
                pltpu.sync_copy(rows.at[0, pl.ds(0, sz)],
                                accum.at[pl.ds(base + off, sz)])
                off += sz
            plsc.subcore_barrier()

            for t in range(_L):
                i_copy(t, p, t).wait()
                g_copy(t % _NB, t).start()

            def body(q, cb):
                for h in range(_NI):
                    j = _NI * q + h
                    b = h % _NB
                    g_copy(b, h).wait()

                    @pl.when(j >= _L)
                    def _():
                        s_copy((h + _NB - _L) % _NB,
                               (h + _NI - _L) % _NI).wait()

                    @pl.when(j + _L < nch)
                    def _():
                        i_copy((h + _L) % _NI, p, j + _L).wait()
                        g_copy((h + _L) % _NB, (h + _L) % _NI).start()

                    w_copy(h, p, j).wait()
                    scale(b, h)
                    s_copy(b, h).start(add=True)

                    @pl.when(j + _NI - 2 < nch)
                    def _():
                        i_copy((h + _NI - 2) % _NI, p, j + _NI - 2).start()
                        w_copy((h + _NI - 2) % _NI, p, j + _NI - 2).start()

                return cb

            lax.fori_loop(0, nch // _NI, body, 0)
            for t in range(_L):
                jt = nch - _L + t
                s_copy(jt % _NB, jt % _NI).wait()
            plsc.subcore_barrier()

            off = 0
            while off < rpt:
                sz = min(ck, rpt - off)
                pltpu.sync_copy(accum.at[pl.ds(base + off, sz)],
                                out_hbm.at[ci, p, pl.ds(base + off, sz)])
                off += sz
            plsc.subcore_barrier()
            return carry

        lax.fori_loop(0, 4, problem, 0)

    return k


def _spmm4(xs, idx4, w4):
    n4, c = xs.shape
    return _spmm4_kernel(n4 // 4, c, idx4.shape[2])(xs, idx4, w4)


def _dense4(gp, w1, b1, w2, n):
    c = gp.shape[-1]
    h1 = w1.shape[1]
    bn = 1000

    def body(g_ref, w1_ref, b1_ref, w2_ref, o_ref):
        g = g_ref[0, 0] + g_ref[1, 0]
        h = jnp.maximum(
            jnp.dot(g, w1_ref[...], preferred_element_type=jnp.float32)
            + b1_ref[0], 0.0)
        o_ref[0] = jnp.dot(h, w2_ref[...], preferred_element_type=jnp.float32)

    return pl.pallas_call(
        body,
        grid=(4, n // bn),
        in_specs=[
            pl.BlockSpec((2, 1, bn, c), lambda k, j: (0, k, j, 0)),
            pl.BlockSpec((c, h1), lambda k, j: (0, 0)),
            pl.BlockSpec((1, h1), lambda k, j: (0, 0)),
            pl.BlockSpec((h1, c), lambda k, j: (0, 0)),
        ],
        out_specs=pl.BlockSpec((1, bn, c), lambda k, j: (k, j, 0)),
        out_shape=jax.ShapeDtypeStruct((4, n, c), jnp.float32),
    )(gp, w1, b1.reshape(1, -1), w2)


def _bias_relu4(sp, b2, n):
    c = sp.shape[-1]
    bn = 1000

    def body(s_ref, b_ref, o_ref):
        o_ref[0] = jnp.maximum(s_ref[0, 0] + s_ref[1, 0] + b_ref[0], 0.0)

    return pl.pallas_call(
        body,
        grid=(4, n // bn),
        in_specs=[
            pl.BlockSpec((2, 1, bn, c), lambda k, j: (0, k, j, 0)),
            pl.BlockSpec((1, c), lambda k, j: (0, 0)),
        ],
        out_specs=pl.BlockSpec((1, bn, c), lambda k, j: (k, j, 0)),
        out_shape=jax.ShapeDtypeStruct((4, n, c), jnp.float32),
    )(sp, b2.reshape(1, -1))


def kernel(feat, shuf_feat, aug1_feat, aug2_feat, adj_index, adj_weight,
           aug1_index, aug1_weight, aug2_index, aug2_weight, W1, b1, W2, b2):
    n = feat.shape[0]
    idx4, w4 = _prep_edges4(adj_index, adj_weight,
                            aug1_index, aug1_weight, n)

    xs1 = jnp.concatenate([feat, shuf_feat, feat, aug2_feat], axis=0)
    g = _spmm4(xs1, idx4, w4)

    p = _dense4(g, W1, b1, W2, n)

    s = _spmm4(p.reshape(4 * n, -1), idx4, w4)

    out = _bias_relu4(s, b2, n)
    return (out[0], out[1], out[2], out[3])
